# baseline (speedup 1.0000x reference)
import functools

import jax
import jax.numpy as jnp
from jax import lax
from jax.experimental import pallas as pl
from jax.experimental.pallas import tpu as pltpu
from jax.experimental.pallas import tpu_sc as plsc

NUM_NODES = 100000
H_DIM = 64
BATCH = 16384
OUT_W = 128

_NC = 2
_NS = 16
_NW = _NC * _NS
_B_PER_W = BATCH // _NW
_C = 4
_CH = _B_PER_W // _C


def _make_gather():
    mesh = plsc.VectorSubcoreMesh(core_axis_name="c", subcore_axis_name="s")

    @functools.partial(
        pl.kernel,
        mesh=mesh,
        compiler_params=pltpu.CompilerParams(
            use_tc_tiling_on_sc=False, needs_layout_passes=False
        ),
        out_type=jax.ShapeDtypeStruct((BATCH, OUT_W), jnp.float32),
        scratch_types=[
            pltpu.VMEM((_B_PER_W, 1), jnp.int32),
            pltpu.VMEM((_B_PER_W,), jnp.int32),
            pltpu.VMEM((_C, _CH, H_DIM), jnp.float32),
            pltpu.SemaphoreType.DMA,
            pltpu.SemaphoreType.DMA,
        ],
    )
    def gather_kernel(idx_hbm, table_hbm, out_hbm, idx_v2, idx_v, rows_v,
                      gsem, wsem):
        wid = lax.axis_index("s") * _NC + lax.axis_index("c")
        base = wid * _B_PER_W
        pltpu.sync_copy(
            idx_hbm.at[pl.ds(base, _B_PER_W), pl.ds(0, 1)], idx_v2
        )
        zeros16 = jnp.zeros((16,), jnp.int32)
        for i in range(_B_PER_W // 16):
            rows = lax.iota(jnp.int32, 16) + (16 * i)
            idx_v[pl.ds(16 * i, 16)] = plsc.load_gather(idx_v2, [rows, zeros16])
        gathers = [
            pltpu.async_copy(
                table_hbm.at[idx_v.at[pl.ds(c * _CH, _CH)]],
                rows_v.at[c],
                gsem,
            )
            for c in range(_C)
        ]
        writebacks = []
        for c in range(_C):
            gathers[c].wait()
            writebacks.append(
                pltpu.async_copy(
                    rows_v.at[c],
                    out_hbm.at[pl.ds(base + c * _CH, _CH), pl.ds(0, H_DIM)],
                    wsem,
                )
            )
        for wb in writebacks:
            wb.wait()

    return gather_kernel


_gather = _make_gather()


def kernel(g, h, table):
    h128 = lax.broadcast_in_dim(h, (BATCH, OUT_W), (0, 1))
    out_padded = _gather(h128, table)
    return out_padded[:, :H_DIM]


# speedup vs baseline: 1.0868x; 1.0868x over previous
import functools

import jax
import jax.numpy as jnp
from jax import lax
from jax.experimental import pallas as pl
from jax.experimental.pallas import tpu as pltpu
from jax.experimental.pallas import tpu_sc as plsc

NUM_NODES = 100000
H_DIM = 64
BATCH = 16384
OUT_W = 128

_NC = 2
_NS = 16
_NW = _NC * _NS
_B_PER_W = BATCH // _NW
_C = 4
_CH = _B_PER_W // _C


def _make_gather():
    mesh = plsc.VectorSubcoreMesh(core_axis_name="c", subcore_axis_name="s")

    @functools.partial(
        pl.kernel,
        mesh=mesh,
        compiler_params=pltpu.CompilerParams(
            use_tc_tiling_on_sc=False, needs_layout_passes=False
        ),
        out_type=jax.ShapeDtypeStruct((BATCH, OUT_W), jnp.float32),
        scratch_types=[
            pltpu.VMEM((_B_PER_W, 1), jnp.int32),
            pltpu.VMEM((_B_PER_W,), jnp.int32),
            pltpu.VMEM((_C, _CH, H_DIM), jnp.float32),
            pltpu.SemaphoreType.DMA,
            pltpu.SemaphoreType.DMA,
        ],
    )
    def gather_kernel(idx_hbm, table_hbm, out_hbm, idx_v2, idx_v, rows_v,
                      gsem, wsem):
        wid = lax.axis_index("s") * _NC + lax.axis_index("c")
        base = wid * _B_PER_W
        pltpu.sync_copy(
            idx_hbm.at[pl.ds(base, _B_PER_W), pl.ds(0, 1)], idx_v2
        )
        zeros16 = jnp.zeros((16,), jnp.int32)
        for i in range(_B_PER_W // 16):
            rows = lax.iota(jnp.int32, 16) + (16 * i)
            idx_v[pl.ds(16 * i, 16)] = plsc.load_gather(idx_v2, [rows, zeros16])
        gathers = [
            pltpu.async_copy(
                table_hbm.at[idx_v.at[pl.ds(c * _CH, _CH)]],
                rows_v.at[c],
                gsem,
            )
            for c in range(_C)
        ]
        writebacks = []
        for c in range(_C):
            gathers[c].wait()
            writebacks.append(
                pltpu.async_copy(
                    rows_v.at[c],
                    out_hbm.at[pl.ds(base + c * _CH, _CH), pl.ds(0, H_DIM)],
                    wsem,
                )
            )
        for wb in writebacks:
            wb.wait()

    return gather_kernel


_gather = _make_gather()


def kernel(g, h, table):
    zeros_row = lax.optimization_barrier(jnp.zeros((1, OUT_W), h.dtype))
    h128 = h + zeros_row
    out_padded = _gather(h128, table)
    return out_padded[:, :H_DIM]


# speedup vs baseline: 1.1120x; 1.0232x over previous
import functools

import jax
import jax.numpy as jnp
from jax import lax
from jax.experimental import pallas as pl
from jax.experimental.pallas import tpu as pltpu
from jax.experimental.pallas import tpu_sc as plsc

NUM_NODES = 100000
H_DIM = 64
BATCH = 16384
OUT_W = 128

_NC = 2
_NS = 16
_NW = _NC * _NS
_B_PER_W = BATCH // _NW
_C = 4
_CH = _B_PER_W // _C


def _make_squeeze():
    mesh = plsc.VectorSubcoreMesh(core_axis_name="c", subcore_axis_name="s")

    @functools.partial(
        pl.kernel,
        mesh=mesh,
        compiler_params=pltpu.CompilerParams(needs_layout_passes=False),
        out_type=jax.ShapeDtypeStruct((BATCH,), jnp.int32),
        scratch_types=[
            pltpu.VMEM((_B_PER_W, 1), jnp.int32),
            pltpu.VMEM((_B_PER_W,), jnp.int32),
        ],
    )
    def squeeze_kernel(h_hbm, out_hbm, col_v, idx_v):
        wid = lax.axis_index("s") * _NC + lax.axis_index("c")
        base = wid * _B_PER_W
        pltpu.sync_copy(h_hbm.at[pl.ds(base, _B_PER_W)], col_v)
        zeros16 = jnp.zeros((16,), jnp.int32)
        for i in range(_B_PER_W // 16):
            rows = lax.iota(jnp.int32, 16) + (16 * i)
            idx_v[pl.ds(16 * i, 16)] = plsc.load_gather(col_v, [rows, zeros16])
        pltpu.sync_copy(idx_v, out_hbm.at[pl.ds(base, _B_PER_W)])

    return squeeze_kernel


def _make_gather():
    mesh = plsc.VectorSubcoreMesh(core_axis_name="c", subcore_axis_name="s")

    @functools.partial(
        pl.kernel,
        mesh=mesh,
        compiler_params=pltpu.CompilerParams(
            use_tc_tiling_on_sc=False, needs_layout_passes=False
        ),
        out_type=jax.ShapeDtypeStruct((BATCH, OUT_W), jnp.float32),
        scratch_types=[
            pltpu.VMEM((_B_PER_W,), jnp.int32),
            pltpu.VMEM((_C, _CH, H_DIM), jnp.float32),
            pltpu.SemaphoreType.DMA,
            pltpu.SemaphoreType.DMA,
        ],
    )
    def gather_kernel(idx_hbm, table_hbm, out_hbm, idx_v, rows_v,
                      gsem, wsem):
        wid = lax.axis_index("s") * _NC + lax.axis_index("c")
        base = wid * _B_PER_W
        pltpu.sync_copy(idx_hbm.at[pl.ds(base, _B_PER_W)], idx_v)
        gathers = [
            pltpu.async_copy(
                table_hbm.at[idx_v.at[pl.ds(c * _CH, _CH)]],
                rows_v.at[c],
                gsem,
            )
            for c in range(_C)
        ]
        writebacks = []
        for c in range(_C):
            gathers[c].wait()
            writebacks.append(
                pltpu.async_copy(
                    rows_v.at[c],
                    out_hbm.at[pl.ds(base + c * _CH, _CH), pl.ds(0, H_DIM)],
                    wsem,
                )
            )
        for wb in writebacks:
            wb.wait()

    return gather_kernel


_squeeze = _make_squeeze()
_gather = _make_gather()


def kernel(g, h, table):
    idx = _squeeze(h)
    out_padded = _gather(idx, table)
    return out_padded[:, :H_DIM]
